# baseline (speedup 1.0000x reference)
import jax
import jax.numpy as jnp
from jax.experimental import pallas as pl
from jax.experimental.pallas import tpu as pltpu

_BB = 64


def _noise_kernel(t_ref, ab_ref, x0_ref, eps_ref, noisy_ref):
    tb = t_ref[...]
    ab_row = ab_ref[...]
    T = ab_row.shape[2]
    onehot = jax.lax.broadcasted_iota(jnp.int32, (tb.shape[0], 1, T), 2) == tb
    abar = jnp.sum(jnp.where(onehot, ab_row, 0.0), axis=2, keepdims=True)
    a = jnp.sqrt(abar)
    b = jnp.sqrt(jnp.maximum(1.0 - abar, 0.0))
    noisy_ref[...] = a * x0_ref[...] + b * eps_ref[...]


def kernel(x0, t, eps, alphabar):
    B, S, D = x0.shape
    T = alphabar.shape[0]
    t3 = t.astype(jnp.int32).reshape(B, 1, 1)
    ab3 = alphabar.reshape(1, 1, T)
    grid = (B // _BB,)
    noisy = pl.pallas_call(
        _noise_kernel,
        grid=grid,
        in_specs=[
            pl.BlockSpec((_BB, 1, 1), lambda i: (i, 0, 0)),
            pl.BlockSpec((1, 1, T), lambda i: (0, 0, 0)),
            pl.BlockSpec((_BB, S, D), lambda i: (i, 0, 0)),
            pl.BlockSpec((_BB, S, D), lambda i: (i, 0, 0)),
        ],
        out_specs=pl.BlockSpec((_BB, S, D), lambda i: (i, 0, 0)),
        out_shape=jax.ShapeDtypeStruct((B, S, D), jnp.float32),
        compiler_params=pltpu.CompilerParams(
            dimension_semantics=("parallel",),
        ),
    )(t3, ab3, x0, eps)
    return noisy, eps


# speedup vs baseline: 1.5594x; 1.5594x over previous
import jax
import jax.numpy as jnp
from jax.experimental import pallas as pl
from jax.experimental.pallas import tpu as pltpu

_CB = 32
_NSPLIT = 4
_NBUF = 6
_RS = _CB // _NSPLIT


def _noise_kernel(t_ref, ab_ref, x_hbm, e_hbm, o_hbm,
                  xb, eb, ob, sx, se, so):
    B = x_hbm.shape[0]
    nchunks = B // _CB
    ab_row = ab_ref[...]
    T = ab_row.shape[1]

    def copies(hbm, buf, sem, c, slot, to_hbm=False):
        out = []
        for j in range(_NSPLIT):
            hslc = hbm.at[pl.ds(c * _CB + j * _RS, _RS), :]
            vslc = buf.at[slot, pl.ds(j * _RS, _RS), :]
            if to_hbm:
                out.append(pltpu.make_async_copy(vslc, hslc, sem.at[slot, j]))
            else:
                out.append(pltpu.make_async_copy(hslc, vslc, sem.at[slot, j]))
        return out

    def start(cps):
        for cp in cps:
            cp.start()

    def wait(cps):
        for cp in cps:
            cp.wait()

    for s in range(_NBUF):
        start(copies(x_hbm, xb, sx, s, s))
        start(copies(e_hbm, eb, se, s, s))

    def body(i, _):
        slot = jax.lax.rem(i, _NBUF)
        wait(copies(x_hbm, xb, sx, i, slot))
        wait(copies(e_hbm, eb, se, i, slot))
        tb = t_ref[pl.ds(i * _CB, _CB), :]
        onehot = jax.lax.broadcasted_iota(jnp.int32, (_CB, T), 1) == tb
        abar = jnp.sum(jnp.where(onehot, ab_row, 0.0), axis=1, keepdims=True)
        a = jnp.sqrt(abar)
        b = jnp.sqrt(jnp.maximum(1.0 - abar, 0.0))

        @pl.when(i >= _NBUF)
        def _():
            wait(copies(o_hbm, ob, so, i - _NBUF, slot, to_hbm=True))

        ob[slot] = a * xb[slot] + b * eb[slot]
        start(copies(o_hbm, ob, so, i, slot, to_hbm=True))

        @pl.when(i + _NBUF < nchunks)
        def _():
            start(copies(x_hbm, xb, sx, i + _NBUF, slot))
            start(copies(e_hbm, eb, se, i + _NBUF, slot))

        return 0

    jax.lax.fori_loop(0, nchunks, body, 0)

    for s in range(_NBUF):
        c = nchunks - _NBUF + s
        wait(copies(o_hbm, ob, so, c, c % _NBUF, to_hbm=True))


def kernel(x0, t, eps, alphabar):
    B, S, D = x0.shape
    SD = S * D
    T = alphabar.shape[0]
    x2 = x0.reshape(B, SD)
    e2 = eps.reshape(B, SD)
    t2 = t.astype(jnp.int32).reshape(B, 1)
    ab2 = alphabar.reshape(1, T)
    noisy = pl.pallas_call(
        _noise_kernel,
        in_specs=[
            pl.BlockSpec(memory_space=pltpu.VMEM),
            pl.BlockSpec(memory_space=pltpu.VMEM),
            pl.BlockSpec(memory_space=pl.ANY),
            pl.BlockSpec(memory_space=pl.ANY),
        ],
        out_specs=pl.BlockSpec(memory_space=pl.ANY),
        out_shape=jax.ShapeDtypeStruct((B, SD), jnp.float32),
        scratch_shapes=[
            pltpu.VMEM((_NBUF, _CB, SD), jnp.float32),
            pltpu.VMEM((_NBUF, _CB, SD), jnp.float32),
            pltpu.VMEM((_NBUF, _CB, SD), jnp.float32),
            pltpu.SemaphoreType.DMA((_NBUF, _NSPLIT)),
            pltpu.SemaphoreType.DMA((_NBUF, _NSPLIT)),
            pltpu.SemaphoreType.DMA((_NBUF, _NSPLIT)),
        ],
    )(t2, ab2, x2, e2)
    return noisy.reshape(B, S, D), eps
